# Initial kernel scaffold; baseline (speedup 1.0000x reference)
import jax
import jax.numpy as jnp
from jax.experimental import pallas as pl


def kernel(x, segment_ids, adduct):
    raise NotImplementedError("write your pallas kernel here")


# speedup vs baseline: 4.6925x; 4.6925x over previous
import functools

import jax
import jax.numpy as jnp
from jax import lax
from jax.experimental import pallas as pl
from jax.experimental.pallas import tpu as pltpu, tpu_sc as plsc

N = 320000
D = 128
B = 2048
D_ADDUCT = 16

NBLK = N // 128
NW = 32
BASE_PER_W = NBLK // NW
EXTRA = NBLK - BASE_PER_W * NW


def _sc_body(x_hbm, ids_hbm, z128_hbm, ones_hbm,
             psum_hbm, pcnt_hbm,
             sums_sp, cnts_sp, xbuf, idbuf, onesbuf):
    cid = lax.axis_index("c")
    sid = lax.axis_index("s")
    wid = sid * 2 + cid

    pltpu.sync_copy(z128_hbm, sums_sp.at[pl.ds(sid * 128, 128)])
    pltpu.sync_copy(z128_hbm, cnts_sp.at[pl.ds(sid * 128, 128)])
    pltpu.sync_copy(ones_hbm, onesbuf)
    plsc.subcore_barrier()

    start = wid * BASE_PER_W + jnp.minimum(wid, EXTRA)
    nblocks = jnp.where(wid < EXTRA, BASE_PER_W + 1, BASE_PER_W)

    def step(i, carry):
        blk = start + i
        pltpu.sync_copy(ids_hbm.at[blk], idbuf)
        pltpu.sync_copy(x_hbm.at[pl.ds(blk * 128, 128)], xbuf)
        pltpu.sync_copy(xbuf, sums_sp.at[idbuf], add=True)
        pltpu.sync_copy(onesbuf, cnts_sp.at[idbuf], add=True)
        return carry

    lax.fori_loop(0, nblocks, step, 0)
    plsc.subcore_barrier()

    out_row = cid * B + sid * 128
    pltpu.sync_copy(sums_sp.at[pl.ds(sid * 128, 128)],
                    psum_hbm.at[pl.ds(out_row, 128)])
    pltpu.sync_copy(cnts_sp.at[pl.ds(sid * 128, 128)],
                    pcnt_hbm.at[pl.ds(out_row, 128)])


_sc_call = functools.partial(
    pl.kernel,
    out_type=(
        jax.ShapeDtypeStruct((2 * B, D), jnp.float32),
        jax.ShapeDtypeStruct((2 * B, D), jnp.float32),
    ),
    mesh=plsc.VectorSubcoreMesh(core_axis_name="c", subcore_axis_name="s"),
    scratch_types=[
        pltpu.VMEM_SHARED((B, D), jnp.float32),
        pltpu.VMEM_SHARED((B, D), jnp.float32),
        pltpu.VMEM((128, D), jnp.float32),
        pltpu.VMEM((128,), jnp.int32),
        pltpu.VMEM((128, D), jnp.float32),
    ],
)(_sc_body)


def _combine_body(ps_ref, pc_ref, ad_ref, o_ref):
    s = ps_ref[0:B, :] + ps_ref[B:2 * B, :]
    c = pc_ref[0:B, :] + pc_ref[B:2 * B, :]
    o_ref[...] = jnp.concatenate([s / jnp.maximum(c, 1.0), ad_ref[...]], axis=1)


_combine = pl.pallas_call(
    _combine_body,
    out_shape=jax.ShapeDtypeStruct((B, D + D_ADDUCT), jnp.float32),
)


def kernel(x, segment_ids, adduct):
    ids2 = segment_ids.reshape(NBLK, 128)
    z128 = jnp.zeros((128, D), jnp.float32)
    ones128 = jnp.ones((128, D), jnp.float32)
    psums, pcnts = _sc_call(x, ids2, z128, ones128)
    return _combine(psums, pcnts, adduct.astype(jnp.float32))


# speedup vs baseline: 6.1766x; 1.3163x over previous
import functools

import jax
import jax.numpy as jnp
from jax import lax
from jax.experimental import pallas as pl
from jax.experimental.pallas import tpu as pltpu, tpu_sc as plsc

N = 320000
D = 128
B = 2048
D_ADDUCT = 16

NBLK = N // 128
NW = 32
PER_W = NBLK // NW
EXTRA = NBLK - PER_W * NW
PAIRS = PER_W // 2


def _sc_body(x_hbm, ids_hbm, z128_hbm, ones_hbm,
             psum_hbm, pcnt_hbm,
             sums_sp, cnts_sp,
             xbuf, idbuf, onesbuf,
             semx0, semx1, semi0, semi1, sems0, sems1, semc0, semc1):
    cid = lax.axis_index("c")
    sid = lax.axis_index("s")
    wid = sid * 2 + cid

    pltpu.sync_copy(z128_hbm, sums_sp.at[pl.ds(sid * 128, 128)])
    pltpu.sync_copy(z128_hbm, cnts_sp.at[pl.ds(sid * 128, 128)])
    pltpu.sync_copy(ones_hbm, onesbuf)
    plsc.subcore_barrier()

    start = wid * PER_W + jnp.minimum(wid, EXTRA)

    def fill(buf, sx, si, blk):
        pltpu.async_copy(x_hbm.at[pl.ds(blk * 128, 128)], xbuf.at[buf], sx)
        pltpu.async_copy(ids_hbm.at[blk], idbuf.at[buf], si)

    def wait_fill(buf, sx, si, blk):
        pltpu.make_async_copy(x_hbm.at[pl.ds(blk * 128, 128)],
                              xbuf.at[buf], sx).wait()
        pltpu.make_async_copy(ids_hbm.at[blk], idbuf.at[buf], si).wait()

    def scatter(buf, ss, sc):
        ds = pltpu.async_copy(xbuf.at[buf], sums_sp.at[idbuf.at[buf]],
                              ss, add=True)
        dc = pltpu.async_copy(onesbuf, cnts_sp.at[idbuf.at[buf]],
                              sc, add=True)
        return ds, dc

    fill(0, semx0, semi0, start)

    def pair(k, carry):
        b0 = start + 2 * k
        b1 = b0 + 1
        wait_fill(0, semx0, semi0, b0)
        fill(1, semx1, semi1, b1)
        ds0, dc0 = scatter(0, sems0, semc0)
        wait_fill(1, semx1, semi1, b1)
        ds1, dc1 = scatter(1, sems1, semc1)
        ds0.wait()
        dc0.wait()

        @pl.when(k + 1 < PAIRS)
        def _():
            fill(0, semx0, semi0, b0 + 2)

        ds1.wait()
        dc1.wait()
        return carry

    lax.fori_loop(0, PAIRS, pair, 0)

    @pl.when(wid < EXTRA)
    def _():
        blk = start + PER_W
        fill(0, semx0, semi0, blk)
        wait_fill(0, semx0, semi0, blk)
        pltpu.sync_copy(xbuf.at[0], sums_sp.at[idbuf.at[0]], add=True)
        pltpu.sync_copy(onesbuf, cnts_sp.at[idbuf.at[0]], add=True)

    plsc.subcore_barrier()

    out_row = cid * B + sid * 128
    pltpu.sync_copy(sums_sp.at[pl.ds(sid * 128, 128)],
                    psum_hbm.at[pl.ds(out_row, 128)])
    pltpu.sync_copy(cnts_sp.at[pl.ds(sid * 128, 128)],
                    pcnt_hbm.at[pl.ds(out_row, 128)])


_sc_call = functools.partial(
    pl.kernel,
    out_type=(
        jax.ShapeDtypeStruct((2 * B, D), jnp.float32),
        jax.ShapeDtypeStruct((2 * B, D), jnp.float32),
    ),
    mesh=plsc.VectorSubcoreMesh(core_axis_name="c", subcore_axis_name="s"),
    scratch_types=[
        pltpu.VMEM_SHARED((B, D), jnp.float32),
        pltpu.VMEM_SHARED((B, D), jnp.float32),
        pltpu.VMEM((2, 128, D), jnp.float32),
        pltpu.VMEM((2, 128), jnp.int32),
        pltpu.VMEM((128, D), jnp.float32),
        pltpu.SemaphoreType.DMA,
        pltpu.SemaphoreType.DMA,
        pltpu.SemaphoreType.DMA,
        pltpu.SemaphoreType.DMA,
        pltpu.SemaphoreType.DMA,
        pltpu.SemaphoreType.DMA,
        pltpu.SemaphoreType.DMA,
        pltpu.SemaphoreType.DMA,
    ],
)(_sc_body)


def _combine_body(ps_ref, pc_ref, ad_ref, o_ref):
    s = ps_ref[0:B, :] + ps_ref[B:2 * B, :]
    c = pc_ref[0:B, :] + pc_ref[B:2 * B, :]
    o_ref[...] = jnp.concatenate([s / jnp.maximum(c, 1.0), ad_ref[...]], axis=1)


_combine = pl.pallas_call(
    _combine_body,
    out_shape=jax.ShapeDtypeStruct((B, D + D_ADDUCT), jnp.float32),
)


def kernel(x, segment_ids, adduct):
    ids2 = segment_ids.reshape(NBLK, 128)
    z128 = jnp.zeros((128, D), jnp.float32)
    ones128 = jnp.ones((128, D), jnp.float32)
    psums, pcnts = _sc_call(x, ids2, z128, ones128)
    return _combine(psums, pcnts, adduct.astype(jnp.float32))


# speedup vs baseline: 8.9607x; 1.4508x over previous
import functools

import jax
import jax.numpy as jnp
from jax import lax
from jax.experimental import pallas as pl
from jax.experimental.pallas import tpu as pltpu, tpu_sc as plsc

N = 320000
D = 128
B = 2048
D_ADDUCT = 16

NBLK = N // 128
NW = 32
PER_W = NBLK // NW
EXTRA = NBLK - PER_W * NW
PAIRS = PER_W // 2


def _sc_body(x_hbm, ids_hbm, z128_hbm, m1_hbm,
             psum_hbm, pcnt_hbm,
             sums_sp, cnts_sp,
             xbuf, idbuf, first_tbl, last_tbl, cntbuf, cbuf, cexp, idx16,
             semx0, semx1, semi0, semi1, sems0, sems1):
    cid = lax.axis_index("c")
    sid = lax.axis_index("s")
    wid = sid * 2 + cid
    i32 = jnp.int32
    iota = lax.iota(i32, 16)

    pltpu.sync_copy(z128_hbm, sums_sp.at[pl.ds(sid * 128, 128)])

    @pl.when(sid == 0)
    def _():
        pltpu.sync_copy(z128_hbm.at[pl.ds(0, 16)], cnts_sp)

    pltpu.sync_copy(m1_hbm, first_tbl)
    pltpu.sync_copy(m1_hbm, last_tbl)
    idx16[...] = iota
    plsc.subcore_barrier()

    start = wid * PER_W + jnp.minimum(wid, EXTRA)

    def fill(buf, sx, si, blk):
        pltpu.async_copy(x_hbm.at[pl.ds(blk * 128, 128)], xbuf.at[buf], sx)
        pltpu.async_copy(ids_hbm.at[blk], idbuf.at[buf], si)

    def wait_fill(buf, sx, si, blk):
        pltpu.make_async_copy(x_hbm.at[pl.ds(blk * 128, 128)],
                              xbuf.at[buf], sx).wait()
        pltpu.make_async_copy(ids_hbm.at[blk], idbuf.at[buf], si).wait()

    def boundaries(buf, blk, prev_last):
        bvec = jnp.full((16,), buf, i32)
        new_last = plsc.load_gather(idbuf, [bvec, jnp.full((16,), 127, i32)])
        for v in range(8):
            p = iota + v * 16
            ids16 = idbuf[buf, pl.ds(v * 16, 16)]
            prv = plsc.load_gather(idbuf, [bvec, jnp.maximum(p - 1, 0)])
            nxt = plsc.load_gather(idbuf, [bvec, jnp.minimum(p + 1, 127)])
            pr = jnp.where(p == 0, prev_last, prv)
            first_m = ids16 != pr
            last_m = (ids16 != nxt) | (p == 127)
            gpos = blk * 128 + p
            hi = lax.shift_right_logical(ids16, 7)
            lo = lax.bitwise_and(ids16, 127)
            plsc.store_scatter(first_tbl, [hi, lo], gpos, mask=first_m)
            plsc.store_scatter(last_tbl, [hi, lo], gpos, mask=last_m)
        return new_last

    fill(0, semx0, semi0, start)

    def pair(k, prev_last):
        b0 = start + 2 * k
        b1 = b0 + 1
        wait_fill(0, semx0, semi0, b0)
        fill(1, semx1, semi1, b1)
        ds0 = pltpu.async_copy(xbuf.at[0], sums_sp.at[idbuf.at[0]],
                               sems0, add=True)
        pl0 = boundaries(0, b0, prev_last)
        wait_fill(1, semx1, semi1, b1)
        ds1 = pltpu.async_copy(xbuf.at[1], sums_sp.at[idbuf.at[1]],
                               sems1, add=True)
        pl1 = boundaries(1, b1, pl0)
        ds0.wait()

        @pl.when(k + 1 < PAIRS)
        def _():
            fill(0, semx0, semi0, b0 + 2)

        ds1.wait()
        return pl1

    prev_last = lax.fori_loop(0, PAIRS, pair, jnp.full((16,), -1, i32))

    @pl.when(wid < EXTRA)
    def _():
        blk = start + PER_W
        fill(0, semx0, semi0, blk)
        wait_fill(0, semx0, semi0, blk)
        pltpu.sync_copy(xbuf.at[0], sums_sp.at[idbuf.at[0]], add=True)
        boundaries(0, blk, prev_last)

    for r in range(16):
        for c in range(8):
            sl = pl.ds(c * 16, 16)
            f = first_tbl[r, sl]
            l = last_tbl[r, sl]
            cnt = jnp.where(f >= 0, (l - f + 1).astype(jnp.float32), 0.0)
            cntbuf[r, sl] = cnt
    pltpu.sync_copy(cntbuf, cnts_sp.at[idx16], add=True)
    plsc.subcore_barrier()

    pltpu.sync_copy(cnts_sp.at[sid], cbuf)

    @pl.loop(0, 128)
    def _expand(j):
        jv = jnp.full((16,), j, i32)
        vec = plsc.load_gather(cbuf, [jv])
        for v in range(8):
            plsc.store_scatter(cexp, [jv, iota + v * 16], vec)

    out_row = cid * B + sid * 128
    pltpu.sync_copy(sums_sp.at[pl.ds(sid * 128, 128)],
                    psum_hbm.at[pl.ds(out_row, 128)])
    pltpu.sync_copy(cexp, pcnt_hbm.at[pl.ds(out_row, 128)])


_sc_call = functools.partial(
    pl.kernel,
    out_type=(
        jax.ShapeDtypeStruct((2 * B, D), jnp.float32),
        jax.ShapeDtypeStruct((2 * B, D), jnp.float32),
    ),
    mesh=plsc.VectorSubcoreMesh(core_axis_name="c", subcore_axis_name="s"),
    compiler_params=pltpu.CompilerParams(needs_layout_passes=False),
    scratch_types=[
        pltpu.VMEM_SHARED((B, D), jnp.float32),
        pltpu.VMEM_SHARED((16, 128), jnp.float32),
        pltpu.VMEM((2, 128, D), jnp.float32),
        pltpu.VMEM((2, 128), jnp.int32),
        pltpu.VMEM((16, 128), jnp.int32),
        pltpu.VMEM((16, 128), jnp.int32),
        pltpu.VMEM((16, 128), jnp.float32),
        pltpu.VMEM((128,), jnp.float32),
        pltpu.VMEM((128, D), jnp.float32),
        pltpu.VMEM((16,), jnp.int32),
        pltpu.SemaphoreType.DMA,
        pltpu.SemaphoreType.DMA,
        pltpu.SemaphoreType.DMA,
        pltpu.SemaphoreType.DMA,
        pltpu.SemaphoreType.DMA,
        pltpu.SemaphoreType.DMA,
    ],
)(_sc_body)


def _combine_body(ps_ref, pc_ref, ad_ref, o_ref):
    s = ps_ref[0:B, :] + ps_ref[B:2 * B, :]
    c = pc_ref[0:B, :] + pc_ref[B:2 * B, :]
    o_ref[...] = jnp.concatenate([s / jnp.maximum(c, 1.0), ad_ref[...]], axis=1)


_combine = pl.pallas_call(
    _combine_body,
    out_shape=jax.ShapeDtypeStruct((B, D + D_ADDUCT), jnp.float32),
)


def kernel(x, segment_ids, adduct):
    ids2 = segment_ids.reshape(NBLK, 128)
    z128 = jnp.zeros((128, D), jnp.float32)
    m1 = jnp.full((16, 128), -1, jnp.int32)
    psums, pcnts = _sc_call(x, ids2, z128, m1)
    return _combine(psums, pcnts, adduct.astype(jnp.float32))
